# speedup vs baseline: 1.3145x; 1.3145x over previous
import jax
import jax.numpy as jnp
from jax import lax
from jax.experimental import pallas as pl
from jax.experimental.pallas import tpu as pltpu
from jax.experimental.pallas import tpu_sc as plsc

VOCAB = 100000
MAX_POS = 2048
DIM = 768
B = 4
S = 2048
N = B * S

_info = plsc.get_sparse_core_info()
NC, NS, L = _info.num_cores, _info.num_subcores, _info.num_lanes
NW = NC * NS
TOK_PER_W = N // NW
CHUNK = 64
NCHUNK = TOK_PER_W // CHUNK
DV = DIM // L


def _body(ids_hbm, wte_hbm, wpe_hbm, out_hbm, idx_v, rows_v, wpe_v, sem):
    wid = lax.axis_index("s") * NC + lax.axis_index("c")
    base = wid * TOK_PER_W
    s0 = lax.rem(base, S)
    pltpu.sync_copy(ids_hbm.at[pl.ds(base, TOK_PER_W)], idx_v)
    for c in range(NCHUNK):
        pltpu.async_copy(
            wte_hbm.at[idx_v.at[pl.ds(c * CHUNK, CHUNK)]], rows_v, sem
        ).wait()
        pltpu.sync_copy(wpe_hbm.at[pl.ds(s0 + c * CHUNK, CHUNK)], wpe_v)

        @pl.loop(0, CHUNK)
        def _(r):
            for j in range(DV):
                sl = pl.ds(j * L, L)
                rows_v[r, sl] = rows_v[r, sl] + wpe_v[r, sl]

        pltpu.sync_copy(rows_v, out_hbm.at[pl.ds(base + c * CHUNK, CHUNK)])


@jax.jit
def _run(ids_flat, wte, wpe):
    mesh = plsc.VectorSubcoreMesh(core_axis_name="c", subcore_axis_name="s")
    return pl.kernel(
        _body,
        mesh=mesh,
        out_type=jax.ShapeDtypeStruct((N, DIM), jnp.float32),
        scratch_types=[
            pltpu.VMEM((TOK_PER_W,), jnp.int32),
            pltpu.VMEM((CHUNK, DIM), jnp.float32),
            pltpu.VMEM((CHUNK, DIM), jnp.float32),
            pltpu.SemaphoreType.DMA,
        ],
    )(ids_flat, wte, wpe)


def kernel(input_ids, wte, wpe):
    ids_flat = input_ids.reshape(N).astype(jnp.int32)
    out = _run(ids_flat, wte, wpe)
    return out.reshape(B, S, DIM)


# speedup vs baseline: 1.3394x; 1.0189x over previous
import jax
import jax.numpy as jnp
from jax import lax
from jax.experimental import pallas as pl
from jax.experimental.pallas import tpu as pltpu
from jax.experimental.pallas import tpu_sc as plsc

VOCAB = 100000
MAX_POS = 2048
DIM = 768
B = 4
S = 2048
N = B * S

_info = plsc.get_sparse_core_info()
NC, NS, L = _info.num_cores, _info.num_subcores, _info.num_lanes
NW = NC * NS
PPW = S // NW
TOK_PER_W = B * PPW
CHUNK = 32
CPB = PPW // CHUNK
NCHUNK = B * CPB
DV = DIM // L


def _body(ids_hbm, wte_hbm, wpe_hbm, out_hbm, idx_v, wpe_v, rows0, rows1,
          g0, g1, st0, st1):
    wid = lax.axis_index("s") * NC + lax.axis_index("c")
    pos0 = wid * PPW
    rows = (rows0, rows1)
    gsem = (g0, g1)
    ssem = (st0, st1)
    pltpu.sync_copy(wpe_hbm.at[pl.ds(pos0, PPW)], wpe_v)
    for b in range(B):
        pltpu.sync_copy(ids_hbm.at[pl.ds(b * S + pos0, PPW)],
                        idx_v.at[pl.ds(b * PPW, PPW)])

    def gather(k):
        buf = k % 2
        return pltpu.async_copy(
            wte_hbm.at[idx_v.at[pl.ds(k * CHUNK, CHUNK)]], rows[buf],
            gsem[buf])

    gathers = [None] * NCHUNK
    stores = [None] * NCHUNK
    gathers[0] = gather(0)
    for k in range(NCHUNK):
        buf = k % 2
        b, co = divmod(k, CPB)
        if k + 1 < NCHUNK:
            if k >= 1:
                stores[k - 1].wait()
            gathers[k + 1] = gather(k + 1)
        gathers[k].wait()

        @pl.loop(0, CHUNK)
        def _(r):
            for j in range(DV):
                sl = pl.ds(j * L, L)
                plsc.addupdate(rows[buf].at[r, sl], wpe_v[co * CHUNK + r, sl])

        stores[k] = pltpu.async_copy(
            rows[buf], out_hbm.at[pl.ds(b * S + pos0 + co * CHUNK, CHUNK)],
            ssem[buf])
    stores[NCHUNK - 2].wait()
    stores[NCHUNK - 1].wait()


@jax.jit
def _run(ids_flat, wte, wpe):
    mesh = plsc.VectorSubcoreMesh(core_axis_name="c", subcore_axis_name="s")
    return pl.kernel(
        _body,
        mesh=mesh,
        out_type=jax.ShapeDtypeStruct((N, DIM), jnp.float32),
        scratch_types=[
            pltpu.VMEM((TOK_PER_W,), jnp.int32),
            pltpu.VMEM((PPW, DIM), jnp.float32),
            pltpu.VMEM((CHUNK, DIM), jnp.float32),
            pltpu.VMEM((CHUNK, DIM), jnp.float32),
            pltpu.SemaphoreType.DMA,
            pltpu.SemaphoreType.DMA,
            pltpu.SemaphoreType.DMA,
            pltpu.SemaphoreType.DMA,
        ],
    )(ids_flat, wte, wpe)


def kernel(input_ids, wte, wpe):
    ids_flat = input_ids.reshape(N).astype(jnp.int32)
    out = _run(ids_flat, wte, wpe)
    return out.reshape(B, S, DIM)
